# baseline (speedup 1.0000x reference)
import jax
import jax.numpy as jnp
from jax import lax
from jax.experimental import pallas as pl
from jax.experimental.pallas import tpu as pltpu

_B = 16
_A = 129
_N = 128
_AP = 136
_S4 = _AP * 4
_DIM = 256
_H = 4
_C = 64
_L = 2
_NEG = -1e9


def _ln(x, g, b, eps=1e-5):
    mu = jnp.mean(x, axis=-1, keepdims=True)
    var = jnp.mean((x - mu) ** 2, axis=-1, keepdims=True)
    return (x - mu) / jnp.sqrt(var + eps) * g + b


def _gnn_body(agents_ref, ego_ref, wn_ref, bn_ref, gn_ref, ben_ref,
              we_ref, beg_ref, ge_ref, bee_ref,
              wl_ref, asrc_ref, adst_ref, gbias_ref,
              wout_ref, bout_ref, out_ref, h_scr, mnb_scr, meg_scr):
    f32 = jnp.float32

    row_id = lax.broadcasted_iota(jnp.int32, (_AP, _N), 0)
    col_id = lax.broadcasted_iota(jnp.int32, (_AP, _N), 1)
    eye_nb = row_id == col_id + 1
    i_col = lax.broadcasted_iota(jnp.int32, (_AP, 1), 0)
    for b in range(_B):
        ag = agents_ref[b * _AP:(b + 1) * _AP, :]
        agT = ag.T
        dx = ag[:, 0:1] - agT[0:1, 1:_A]
        dy = ag[:, 1:2] - agT[1:2, 1:_A]
        dist = jnp.sqrt(dx * dx + dy * dy + 1e-12)
        mask = ((dist < 50.0) & (~eye_nb)) | eye_nb
        mnb = jnp.where(mask, 0.0, _NEG).astype(f32)
        mnb_scr[b * _S4:(b + 1) * _S4, :] = jnp.concatenate(
            [mnb, mnb, mnb, mnb], axis=0)
        dx0 = ag[:, 0:1] - ag[0:1, 0:1]
        dy0 = ag[:, 1:2] - ag[0:1, 1:2]
        dist0 = jnp.sqrt(dx0 * dx0 + dy0 * dy0 + 1e-12)
        mask0 = (dist0 < 50.0) | (i_col == 0)
        meg = jnp.where(mask0, 0.0, _NEG).astype(f32)
        meg_scr[b * _S4:(b + 1) * _S4, :] = jnp.concatenate(
            [meg, meg, meg, meg], axis=0)

    ag_all = agents_ref[...]
    hn = jnp.maximum(jnp.dot(ag_all, wn_ref[...],
                             preferred_element_type=f32) + bn_ref[...], 0.0)
    h_scr[...] = _ln(hn, gn_ref[...], ben_ref[...])
    he = jnp.maximum(jnp.dot(ego_ref[...], we_ref[...],
                             preferred_element_type=f32) + beg_ref[...], 0.0)
    he = _ln(he, ge_ref[...], bee_ref[...])
    for b in range(_B):
        h_scr[b * _AP:b * _AP + 1, :] = he[b:b + 1, :]

    for l in range(_L):
        h_all = h_scr[...]
        x_all = jnp.dot(h_all, wl_ref[l], preferred_element_type=f32)
        a_dst = jnp.dot(x_all, adst_ref[l], preferred_element_type=f32)
        a_srcT = jnp.dot(x_all, asrc_ref[l],
                         preferred_element_type=f32).T
        for b in range(_B):
            sl = slice(b * _AP, (b + 1) * _AP)
            xb = x_all[sl]
            x_nb = xb[1:_A, :]
            ad = a_dst[sl]
            lg_nb = jnp.concatenate(
                [ad[:, h:h + 1] + a_srcT[h:h + 1, b * _AP + 1:b * _AP + _A]
                 for h in range(_H)], axis=0)
            lg_eg = jnp.concatenate(
                [ad[:, h:h + 1] + a_srcT[h:h + 1, b * _AP:b * _AP + 1]
                 for h in range(_H)], axis=0)
            lg_nb = jnp.maximum(lg_nb, 0.2 * lg_nb) \
                + mnb_scr[b * _S4:(b + 1) * _S4, :]
            lg_eg = jnp.maximum(lg_eg, 0.2 * lg_eg) \
                + meg_scr[b * _S4:(b + 1) * _S4, :]
            m = jnp.maximum(jnp.max(lg_nb, axis=1, keepdims=True), lg_eg)
            e_nb = jnp.exp(lg_nb - m)
            e_eg = jnp.exp(lg_eg - m)
            sinv = 1.0 / (jnp.sum(e_nb, axis=1, keepdims=True) + e_eg)
            cols = []
            for h in range(_H):
                hsl = slice(h * _AP, (h + 1) * _AP)
                oh = jnp.dot(e_nb[hsl], x_nb[:, h * _C:(h + 1) * _C],
                             preferred_element_type=f32)
                oh = oh + e_eg[hsl] * xb[0:1, h * _C:(h + 1) * _C]
                cols.append(oh * sinv[hsl])
            ob = jnp.concatenate(cols, axis=1)
            h_scr[sl, :] = jnp.maximum(ob + gbias_ref[l:l + 1, :], 0.0)

    out_ref[...] = jnp.dot(h_scr[...], wout_ref[...],
                           preferred_element_type=f32) + bout_ref[...]


@jax.jit
def kernel(ego_agent_past, neighbor_agents_past, W_node, b_node, g_node,
           be_node, W_ego, b_ego, g_ego, be_ego, gat_W, gat_att_src,
           gat_att_dst, gat_bias, W_out, b_out):
    ego_last = ego_agent_past[:, -1, :5]
    nb_last = neighbor_agents_past[:, :, -1, :5]
    agents = jnp.concatenate([ego_last[:, None, :], nb_last], axis=1)
    agents = jnp.pad(agents, ((0, 0), (0, _AP - _A), (0, 0)))
    agents = agents.reshape(_B * _AP, 5)

    Wn = W_node[:5]
    We = W_ego[:5]

    L, dim, H, C = gat_W.shape
    Wl = gat_W.reshape(L, dim, H * C)
    eyeH = jnp.eye(H, dtype=gat_W.dtype)
    Asrc = (gat_att_src[:, :, :, None] * eyeH[None, :, None, :]
            ).reshape(L, H * C, H)
    Adst = (gat_att_dst[:, :, :, None] * eyeH[None, :, None, :]
            ).reshape(L, H * C, H)

    row = lambda v: v.reshape(1, -1)

    out = pl.pallas_call(
        _gnn_body,
        out_shape=jax.ShapeDtypeStruct((_B * _AP, _DIM), jnp.float32),
        scratch_shapes=[
            pltpu.VMEM((_B * _AP, _DIM), jnp.float32),
            pltpu.VMEM((_B * _S4, _N), jnp.float32),
            pltpu.VMEM((_B * _S4, 1), jnp.float32),
        ],
    )(agents, ego_last, Wn, row(b_node), row(g_node), row(be_node),
      We, row(b_ego), row(g_ego), row(be_ego),
      Wl, Asrc, Adst, gat_bias,
      W_out, row(b_out))
    return out.reshape(_B, _AP, _DIM)[:, :_A, :]


# speedup vs baseline: 1.3286x; 1.3286x over previous
import jax
import jax.numpy as jnp
from jax import lax
from jax.experimental import pallas as pl
from jax.experimental.pallas import tpu as pltpu

_B = 16
_A = 129
_AP = 136
_S4 = _AP * 4
_DIM = 256
_H = 4
_C = 64
_L = 2
_NEG = -1e9


def _ln(x, g, b, eps=1e-5):
    mu = jnp.mean(x, axis=-1, keepdims=True)
    var = jnp.mean((x - mu) ** 2, axis=-1, keepdims=True)
    return (x - mu) / jnp.sqrt(var + eps) * g + b


def _gnn_body(agents_ref, ego_ref, wn_ref, bn_ref, gn_ref, ben_ref,
              we_ref, beg_ref, ge_ref, bee_ref,
              wl_ref, asrc_ref, adst_ref, gbias_ref,
              wout_ref, bout_ref, out_ref, h_scr, madd_scr):
    f32 = jnp.float32

    row_id = lax.broadcasted_iota(jnp.int32, (_AP, _AP), 0)
    col_id = lax.broadcasted_iota(jnp.int32, (_AP, _AP), 1)
    eye_m = row_id == col_id
    col_ok = col_id < _A
    for b in range(_B):
        ag = agents_ref[b * _AP:(b + 1) * _AP, :]
        agT = ag.T
        dx = ag[:, 0:1] - agT[0:1, :]
        dy = ag[:, 1:2] - agT[1:2, :]
        dist = jnp.sqrt(dx * dx + dy * dy + 1e-12)
        mask = ((((dist < 50.0) & (~eye_m)) | eye_m) & col_ok)
        madd = jnp.where(mask, 0.0, _NEG).astype(f32)
        madd4 = jnp.concatenate([madd, madd, madd, madd], axis=0)
        madd_scr[b * _S4:(b + 1) * _S4, :] = madd4

    ag_all = agents_ref[...]
    hn = jnp.maximum(jnp.dot(ag_all, wn_ref[...],
                             preferred_element_type=f32) + bn_ref[...], 0.0)
    h_scr[...] = _ln(hn, gn_ref[...], ben_ref[...])
    he = jnp.maximum(jnp.dot(ego_ref[...], we_ref[...],
                             preferred_element_type=f32) + beg_ref[...], 0.0)
    he = _ln(he, ge_ref[...], bee_ref[...])
    for b in range(_B):
        h_scr[b * _AP:b * _AP + 1, :] = he[b:b + 1, :]

    lane8 = lax.broadcasted_iota(jnp.int32, (1, 8), 1)

    for l in range(_L):
        h_all = h_scr[...]
        x_all = jnp.dot(h_all, wl_ref[l], preferred_element_type=f32)
        a_dst = jnp.dot(x_all, adst_ref[l], preferred_element_type=f32)
        a_srcT = jnp.dot(x_all, asrc_ref[l],
                         preferred_element_type=f32).T
        for b in range(_B):
            sl = slice(b * _AP, (b + 1) * _AP)
            xb = x_all[sl]
            ad = a_dst[sl]
            p = jnp.concatenate(
                [jnp.where(lane8 == h, ad[:, h:h + 1], 0.0)
                 + jnp.where(lane8 == 4 + h, 1.0, 0.0) for h in range(_H)],
                axis=0)
            q = jnp.concatenate(
                [jnp.ones((_H, _AP), f32), a_srcT[:, sl]], axis=0)
            lg = jnp.dot(p, q, preferred_element_type=f32)
            lg = jnp.maximum(lg, 0.2 * lg) \
                + madd_scr[b * _S4:(b + 1) * _S4, :]
            m = jnp.max(lg, axis=1, keepdims=True)
            e = jnp.exp(lg - m)
            s = jnp.sum(e, axis=1, keepdims=True)
            e = e / s
            ob = jnp.concatenate(
                [jnp.dot(e[h * _AP:(h + 1) * _AP, :],
                         xb[:, h * _C:(h + 1) * _C],
                         preferred_element_type=f32) for h in range(_H)],
                axis=1)
            h_scr[sl, :] = jnp.maximum(ob + gbias_ref[l:l + 1, :], 0.0)

    out_ref[...] = jnp.dot(h_scr[...], wout_ref[...],
                           preferred_element_type=f32) + bout_ref[...]


@jax.jit
def kernel(ego_agent_past, neighbor_agents_past, W_node, b_node, g_node,
           be_node, W_ego, b_ego, g_ego, be_ego, gat_W, gat_att_src,
           gat_att_dst, gat_bias, W_out, b_out):
    ego_last = ego_agent_past[:, -1, :5]
    nb_last = neighbor_agents_past[:, :, -1, :5]
    agents = jnp.concatenate([ego_last[:, None, :], nb_last], axis=1)
    agents = jnp.pad(agents, ((0, 0), (0, _AP - _A), (0, 0)))
    agents = agents.reshape(_B * _AP, 5)

    Wn = W_node[:5]
    We = W_ego[:5]

    L, dim, H, C = gat_W.shape
    Wl = gat_W.reshape(L, dim, H * C)
    eyeH = jnp.eye(H, dtype=gat_W.dtype)
    Asrc = (gat_att_src[:, :, :, None] * eyeH[None, :, None, :]
            ).reshape(L, H * C, H)
    Adst = (gat_att_dst[:, :, :, None] * eyeH[None, :, None, :]
            ).reshape(L, H * C, H)

    row = lambda v: v.reshape(1, -1)

    out = pl.pallas_call(
        _gnn_body,
        out_shape=jax.ShapeDtypeStruct((_B * _AP, _DIM), jnp.float32),
        scratch_shapes=[
            pltpu.VMEM((_B * _AP, _DIM), jnp.float32),
            pltpu.VMEM((_B * _S4, _AP), jnp.float32),
        ],
    )(agents, ego_last, Wn, row(b_node), row(g_node), row(be_node),
      We, row(b_ego), row(g_ego), row(be_ego),
      Wl, Asrc, Adst, gat_bias,
      W_out, row(b_out))
    return out.reshape(_B, _AP, _DIM)[:, :_A, :]
